# baseline (speedup 1.0000x reference)
import functools

import jax
import jax.numpy as jnp
from jax import lax
from jax.experimental import pallas as pl
from jax.experimental.pallas import tpu as pltpu
from jax.experimental.pallas import tpu_sc as plsc

_NC, _NS, _L = 2, 16, 16
_NW = _NC * _NS
NP = 100352
R_BLK = 3200


def _edge_mlp_body(gd_ref, gs_ref, c_ref, w1a, w1b, w1c, b1, w2, b2, w3, b3,
                   o_ref, *, final):
    def nt(w, x):
        return lax.dot_general(w[...], x[...], (((1,), (1,)), ((), ())),
                               preferred_element_type=jnp.float32)

    h = nt(w1a, gd_ref) + nt(w1b, gs_ref) + nt(w1c, c_ref) + b1[...]
    h = jnp.maximum(h, 0.0)
    h = jnp.dot(w2[...], h, preferred_element_type=jnp.float32) + b2[...]
    h = jnp.maximum(h, 0.0)
    o = jnp.dot(w3[...], h, preferred_element_type=jnp.float32) + b3[...]
    if final:
        o_ref[...] = jax.nn.sigmoid(o)
    else:
        o_ref[...] = o.T


def _bd(w, slots_in, slots_out):
    wp = jnp.zeros((slots_in, slots_out), jnp.float32)
    wp = wp.at[:w.shape[0], :w.shape[1]].set(w)
    return jnp.kron(jnp.eye(16, dtype=jnp.float32), wp.T)


def _edge_mlp(gd, gs, c, params, da, dc, dout, *, final=False):
    W1, b1, W2, b2, W3, b3 = params
    G = gd.shape[0]
    H = W1.shape[1]
    grid = G // R_BLK
    w1a = _bd(W1[:da], 8, H)
    w1b = _bd(W1[da:2 * da], 8, H)
    w1c = _bd(W1[2 * da:], 8, H)
    w2 = jnp.kron(jnp.eye(16, dtype=jnp.float32), W2.T)
    mo = 16 if final else 128
    if final:
        w3 = jnp.kron(jnp.eye(16, dtype=jnp.float32), W3.T)
        b3v = jnp.tile(b3, 16)[:, None]
    else:
        w3 = _bd(W3, H, 8)
        b3v = jnp.tile(jnp.pad(b3, (0, 8 - dout)), 16)[:, None]
    b1v = jnp.tile(b1, 16)[:, None]
    b2v = jnp.tile(b2, 16)[:, None]

    def gm(d1):
        return pl.BlockSpec((R_BLK, d1), lambda i: (i, 0))

    def wm(shape):
        return pl.BlockSpec(shape, lambda i: (0, 0))

    if final:
        out_spec = pl.BlockSpec((16, R_BLK), lambda i: (0, i))
        out_shape = jax.ShapeDtypeStruct((16, G), jnp.float32)
    else:
        out_spec = gm(128)
        out_shape = jax.ShapeDtypeStruct((G, 128), jnp.float32)

    return pl.pallas_call(
        functools.partial(_edge_mlp_body, final=final),
        grid=(grid,),
        in_specs=[
            gm(128), gm(128), gm(128),
            wm((16 * H, 128)), wm((16 * H, 128)), wm((16 * H, 128)),
            wm((16 * H, 1)),
            wm((16 * H, 16 * H)), wm((16 * H, 1)),
            wm((mo, 16 * H)), wm((mo, 1)),
        ],
        out_specs=out_spec,
        out_shape=out_shape,
    )(gd, gs, c, w1a, w1b, w1c, b1v, w2, b2v, w3, b3v)


def _node_mlp_body(x_ref, p0_ref, p1_ref, w1a, w1b, b1, w2, b2, w3, b3,
                   o_ref, *, din_x, d_aggr, dout):
    a = x_ref[...][:, :din_x]
    g = (p0_ref[...] + p1_ref[...])[:, :d_aggr]
    h = (jnp.dot(a, w1a[...], preferred_element_type=jnp.float32)
         + jnp.dot(g, w1b[...], preferred_element_type=jnp.float32)
         + b1[...])
    h = jnp.maximum(h, 0.0)
    h = jnp.dot(h, w2[...], preferred_element_type=jnp.float32) + b2[...]
    h = jnp.maximum(h, 0.0)
    o = jnp.dot(h, w3[...], preferred_element_type=jnp.float32) + b3[...]
    o_ref[...] = jnp.concatenate(
        [o, jnp.zeros((o.shape[0], 8 - dout), jnp.float32)], axis=1)


def _node_mlp(xprev, p0, p1, params, din_x, d_aggr, dout):
    W1, b1, W2, b2, W3, b3 = params
    H = W1.shape[1]
    BN = 6272
    grid = NP // BN

    def rm(d1):
        return pl.BlockSpec((BN, d1), lambda i: (i, 0))

    def wm(shape):
        return pl.BlockSpec(shape, lambda i: (0, 0))

    return pl.pallas_call(
        functools.partial(_node_mlp_body, din_x=din_x, d_aggr=d_aggr,
                          dout=dout),
        grid=(grid,),
        in_specs=[
            rm(8), rm(8), rm(8),
            wm((din_x, H)), wm((d_aggr, H)), wm((1, H)),
            wm((H, H)), wm((1, H)),
            wm((H, dout)), wm((1, dout)),
        ],
        out_specs=rm(8),
        out_shape=jax.ShapeDtypeStruct((NP, 8), jnp.float32),
    )(xprev, p0, p1, W1[:din_x], W1[din_x:], b1[None, :],
      W2, b2[None, :], W3, b3[None, :])


def _sc_gather(table, src, dst, ea_t=None, zeros_flat=None):
    E = src.shape[0]
    Dp = table.shape[1]
    K = 2048
    n_chunks = E // K
    n_iter = (n_chunks + 2 * _NW - 1) // (2 * _NW)
    mesh = plsc.VectorSubcoreMesh(core_axis_name="c", subcore_axis_name="s")
    out_type = (jax.ShapeDtypeStruct((E // K, K, Dp), jnp.float32),
                jax.ShapeDtypeStruct((E // K, K, Dp), jnp.float32))
    scratch = ([pltpu.VMEM((K,), jnp.int32)] * 4
               + [pltpu.VMEM((K, Dp), jnp.float32)] * 4
               + [pltpu.SemaphoreType.DMA] * 12)

    def body(table_h, src_h, dst_h, gd_h, gs_h,
             dv0, sv0, dv1, sv1, rd0, rs0, rd1, rs1, *sems):
        wid = lax.axis_index("s") * _NC + lax.axis_index("c")
        bufs = ((dv0, sv0, rd0, rs0, sems[0:4]),
                (dv1, sv1, rd1, rs1, sems[4:8]))
        wsems = (sems[8:10], sems[10:12])

        def chunk(i, carry):
            cs = [(2 * i + b) * _NW + wid for b in range(2)]
            idx_cps = [None, None]
            g_cps = [None, None]
            w_cps = [None, None]
            for b in range(2):
                dv, sv, rd, rs, ss = bufs[b]

                @pl.when(cs[b] < n_chunks)
                def _(b=b, dv=dv, sv=sv, ss=ss):
                    idx_cps[b] = (
                        pltpu.async_copy(dst_h.at[cs[b]], dv, ss[0]),
                        pltpu.async_copy(src_h.at[cs[b]], sv, ss[1]))
            for b in range(2):
                dv, sv, rd, rs, ss = bufs[b]

                @pl.when(cs[b] < n_chunks)
                def _(b=b, dv=dv, sv=sv, rd=rd, rs=rs, ss=ss):
                    idx_cps[b][0].wait()
                    idx_cps[b][1].wait()
                    g_cps[b] = (
                        pltpu.async_copy(table_h.at[dv], rd, ss[2]),
                        pltpu.async_copy(table_h.at[sv], rs, ss[3]))
            for b in range(2):
                dv, sv, rd, rs, ss = bufs[b]

                @pl.when(cs[b] < n_chunks)
                def _(b=b, rd=rd, rs=rs):
                    g_cps[b][0].wait()
                    g_cps[b][1].wait()
                    w_cps[b] = (
                        pltpu.async_copy(rd, gd_h.at[cs[b]], wsems[b][0]),
                        pltpu.async_copy(rs, gs_h.at[cs[b]], wsems[b][1]))
            for b in range(2):
                @pl.when(cs[b] < n_chunks)
                def _(b=b):
                    w_cps[b][0].wait()
                    w_cps[b][1].wait()
            return carry

        lax.fori_loop(0, n_iter, chunk, 0)

    C = n_chunks
    return pl.kernel(
        body, out_type=out_type, mesh=mesh, scratch_types=scratch,
        compiler_params=pltpu.CompilerParams(use_tc_tiling_on_sc=False,
                                             needs_layout_passes=False),
    )(table, src.reshape(C, K), dst.reshape(C, K))


def _sc_scatter(evals, dst, zeros_nd):
    E, D = evals.shape
    NPt = zeros_nd.shape[0]
    K = 2048
    n_chunks = E // K
    n_iter = (n_chunks + _NW - 1) // _NW
    NPS = NPt // _NS
    mesh = plsc.VectorSubcoreMesh(core_axis_name="c", subcore_axis_name="s")
    scratch = [pltpu.VMEM((K,), jnp.int32),
               pltpu.VMEM((K, D), jnp.float32),
               pltpu.VMEM_SHARED((NPt, D), jnp.float32)]

    def body(ev_h, dst_h, z_h, out_h, dstv, rows, shared):
        cid = lax.axis_index("c")
        sid = lax.axis_index("s")
        wid = sid * _NC + cid
        pltpu.sync_copy(z_h.at[pl.ds(sid * NPS, NPS)],
                        shared.at[pl.ds(sid * NPS, NPS)])
        plsc.subcore_barrier()

        def chunk(i, carry):
            c = i * _NW + wid

            @pl.when(c < n_chunks)
            def _():
                pltpu.sync_copy(dst_h.at[c], dstv)
                pltpu.sync_copy(ev_h.at[c], rows)
                pltpu.sync_copy(rows, shared.at[dstv], add=True)

            return carry

        lax.fori_loop(0, n_iter, chunk, 0)
        plsc.subcore_barrier()
        pltpu.sync_copy(shared.at[pl.ds(sid * NPS, NPS)],
                        out_h.at[cid, pl.ds(sid * NPS, NPS)])

    return pl.kernel(
        body,
        out_type=jax.ShapeDtypeStruct((_NC, NPt, D), jnp.float32),
        mesh=mesh, scratch_types=scratch,
        compiler_params=pltpu.CompilerParams(use_tc_tiling_on_sc=False),
    )(evals.reshape(E // K, K, D), dst.reshape(E // K, K), zeros_nd)


def kernel(x, edge_index, edge_attr,
           in1_r1_0, in1_r1_1, in1_r1_2, in1_r1_3, in1_r1_4, in1_r1_5,
           in1_o_0, in1_o_1, in1_o_2, in1_o_3, in1_o_4, in1_o_5,
           in2_r1_0, in2_r1_1, in2_r1_2, in2_r1_3, in2_r1_4, in2_r1_5,
           in2_o_0, in2_o_1, in2_o_2, in2_o_3, in2_o_4, in2_o_5,
           r2_0, r2_1, r2_2, r2_3, r2_4, r2_5):
    in1_r1 = [in1_r1_0, in1_r1_1, in1_r1_2, in1_r1_3, in1_r1_4, in1_r1_5]
    in1_o = [in1_o_0, in1_o_1, in1_o_2, in1_o_3, in1_o_4, in1_o_5]
    in2_r1 = [in2_r1_0, in2_r1_1, in2_r1_2, in2_r1_3, in2_r1_4, in2_r1_5]
    in2_o = [in2_o_0, in2_o_1, in2_o_2, in2_o_3, in2_o_4, in2_o_5]
    r2 = [r2_0, r2_1, r2_2, r2_3, r2_4, r2_5]
    src = edge_index[0]
    dst = edge_index[1]
    N = x.shape[0]
    E = src.shape[0]

    xp = jnp.zeros((NP, 8), jnp.float32).at[:N, :3].set(x)
    z8 = jnp.zeros((NP, 8), jnp.float32)

    def g(a):
        return a.reshape(E // 16, 128)

    gd, gs = _sc_gather(xp, src, dst)
    ea_g = jnp.zeros((E, 8), jnp.float32).at[:, :3].set(edge_attr)
    ea_g = ea_g.reshape(E // 16, 128)
    e1 = _edge_mlp(g(gd), g(gs), ea_g, in1_r1, 3, 3, 5)
    P1 = _sc_scatter(e1.reshape(E, 8), dst, z8)
    x1p = _node_mlp(xp, P1[0], P1[1], in1_o, 3, 5, 5)

    gd, gs = _sc_gather(x1p, src, dst)
    e2 = _edge_mlp(g(gd), g(gs), e1, in2_r1, 5, 5, 7)
    P2 = _sc_scatter(e2.reshape(E, 8), dst, z8)
    x2p = _node_mlp(x1p, P2[0], P2[1], in2_o, 5, 7, 7)

    gd, gs = _sc_gather(x2p, src, dst)
    o16 = _edge_mlp(g(gd), g(gs), e2, r2, 7, 7, 1, final=True)
    return o16.T.reshape(E, 1)


# speedup vs baseline: 2.4295x; 2.4295x over previous
import functools

import jax
import jax.numpy as jnp
from jax import lax
from jax.experimental import pallas as pl
from jax.experimental.pallas import tpu as pltpu
from jax.experimental.pallas import tpu_sc as plsc

_NC, _NS, _L = 2, 16, 16
_NW = _NC * _NS
NP = 100352
R_BLK = 3200


def _edge_mlp_body(gd_ref, gs_ref, c_ref, w1a, w1b, w1c, b1, w2, b2, w3, b3,
                   o_ref, *, final):
    def nt(w, x):
        return lax.dot_general(w[...], x[...], (((1,), (1,)), ((), ())),
                               preferred_element_type=jnp.float32)

    h = nt(w1a, gd_ref) + nt(w1b, gs_ref) + nt(w1c, c_ref) + b1[...]
    h = jnp.maximum(h, 0.0)
    h = jnp.dot(w2[...], h, preferred_element_type=jnp.float32) + b2[...]
    h = jnp.maximum(h, 0.0)
    o = jnp.dot(w3[...], h, preferred_element_type=jnp.float32) + b3[...]
    if final:
        o_ref[...] = jax.nn.sigmoid(o)
    else:
        o_ref[...] = o.T


def _bd(w, slots_in, slots_out):
    wp = jnp.zeros((slots_in, slots_out), jnp.float32)
    wp = wp.at[:w.shape[0], :w.shape[1]].set(w)
    return jnp.kron(jnp.eye(16, dtype=jnp.float32), wp.T)


def _edge_mlp(gd, gs, c, params, da, dc, dout, *, final=False):
    W1, b1, W2, b2, W3, b3 = params
    G = gd.shape[0]
    H = W1.shape[1]
    grid = G // R_BLK
    w1a = _bd(W1[:da], 8, H)
    w1b = _bd(W1[da:2 * da], 8, H)
    w1c = _bd(W1[2 * da:], 8, H)
    w2 = jnp.kron(jnp.eye(16, dtype=jnp.float32), W2.T)
    mo = 16 if final else 128
    if final:
        w3 = jnp.kron(jnp.eye(16, dtype=jnp.float32), W3.T)
        b3v = jnp.tile(b3, 16)[:, None]
    else:
        w3 = _bd(W3, H, 8)
        b3v = jnp.tile(jnp.pad(b3, (0, 8 - dout)), 16)[:, None]
    b1v = jnp.tile(b1, 16)[:, None]
    b2v = jnp.tile(b2, 16)[:, None]

    def gm(d1):
        return pl.BlockSpec((R_BLK, d1), lambda i: (i, 0))

    def wm(shape):
        return pl.BlockSpec(shape, lambda i: (0, 0))

    if final:
        out_spec = pl.BlockSpec((16, R_BLK), lambda i: (0, i))
        out_shape = jax.ShapeDtypeStruct((16, G), jnp.float32)
    else:
        out_spec = gm(128)
        out_shape = jax.ShapeDtypeStruct((G, 128), jnp.float32)

    return pl.pallas_call(
        functools.partial(_edge_mlp_body, final=final),
        grid=(grid,),
        in_specs=[
            gm(128), gm(128), gm(128),
            wm((16 * H, 128)), wm((16 * H, 128)), wm((16 * H, 128)),
            wm((16 * H, 1)),
            wm((16 * H, 16 * H)), wm((16 * H, 1)),
            wm((mo, 16 * H)), wm((mo, 1)),
        ],
        out_specs=out_spec,
        out_shape=out_shape,
    )(gd, gs, c, w1a, w1b, w1c, b1v, w2, b2v, w3, b3v)


def _node_mlp_body(x_ref, p0_ref, p1_ref, w1a, w1b, b1, w2, b2, w3, b3,
                   o_ref, *, din_x, d_aggr, dout):
    a = x_ref[...][:, :din_x]
    g = (p0_ref[...] + p1_ref[...])[:, :d_aggr]
    h = (jnp.dot(a, w1a[...], preferred_element_type=jnp.float32)
         + jnp.dot(g, w1b[...], preferred_element_type=jnp.float32)
         + b1[...])
    h = jnp.maximum(h, 0.0)
    h = jnp.dot(h, w2[...], preferred_element_type=jnp.float32) + b2[...]
    h = jnp.maximum(h, 0.0)
    o = jnp.dot(h, w3[...], preferred_element_type=jnp.float32) + b3[...]
    o_ref[...] = jnp.concatenate(
        [o, jnp.zeros((o.shape[0], 8 - dout), jnp.float32)], axis=1)


def _node_mlp(xprev, p0, p1, params, din_x, d_aggr, dout):
    W1, b1, W2, b2, W3, b3 = params
    H = W1.shape[1]
    BN = 6272
    grid = NP // BN

    def rm(d1):
        return pl.BlockSpec((BN, d1), lambda i: (i, 0))

    def wm(shape):
        return pl.BlockSpec(shape, lambda i: (0, 0))

    return pl.pallas_call(
        functools.partial(_node_mlp_body, din_x=din_x, d_aggr=d_aggr,
                          dout=dout),
        grid=(grid,),
        in_specs=[
            rm(8), rm(8), rm(8),
            wm((din_x, H)), wm((d_aggr, H)), wm((1, H)),
            wm((H, H)), wm((1, H)),
            wm((H, dout)), wm((1, dout)),
        ],
        out_specs=rm(8),
        out_shape=jax.ShapeDtypeStruct((NP, 8), jnp.float32),
    )(xprev, p0, p1, W1[:din_x], W1[din_x:], b1[None, :],
      W2, b2[None, :], W3, b3[None, :])


def _sc_gather(table, src, dst, ea_t=None, zeros_flat=None):
    E = src.shape[0]
    Dp = table.shape[1]
    K = 2048
    n_chunks = E // K
    n_iter = (n_chunks + 2 * _NW - 1) // (2 * _NW)
    has_ea = ea_t is not None
    mesh = plsc.VectorSubcoreMesh(core_axis_name="c", subcore_axis_name="s")
    out_type = [jax.ShapeDtypeStruct((E // K, K, Dp), jnp.float32),
                jax.ShapeDtypeStruct((E // K, K, Dp), jnp.float32)]
    scratch = ([pltpu.VMEM((K,), jnp.int32)] * 4
               + [pltpu.VMEM((K, Dp), jnp.float32)] * 4
               + [pltpu.SemaphoreType.DMA] * 12)
    if has_ea:
        dea = ea_t.shape[0]
        out_type.append(jax.ShapeDtypeStruct((E // K, K * 8), jnp.float32))
        scratch += [pltpu.VMEM((dea, K), jnp.float32),
                    pltpu.VMEM((K * 8,), jnp.float32)]

    def body(*refs):
        if has_ea:
            (table_h, src_h, dst_h, ea_h, z_h, gd_h, gs_h, ea8_h,
             dv0, sv0, dv1, sv1, rd0, rs0, rd1, rs1, *rest) = refs
            sems, fa, eab = rest[:12], rest[12], rest[13]
        else:
            (table_h, src_h, dst_h, gd_h, gs_h,
             dv0, sv0, dv1, sv1, rd0, rs0, rd1, rs1, *sems) = refs
        wid = lax.axis_index("s") * _NC + lax.axis_index("c")
        iota = lax.broadcasted_iota(jnp.int32, (_L,), 0)
        bufs = ((dv0, sv0, rd0, rs0, sems[0:4]),
                (dv1, sv1, rd1, rs1, sems[4:8]))
        wsems = (sems[8:10], sems[10:12])
        if has_ea:
            pltpu.sync_copy(z_h, eab)

        def chunk(i, carry):
            cs = [(2 * i + b) * _NW + wid for b in range(2)]
            idx_cps = [None, None]
            g_cps = [None, None]
            w_cps = [None, None]
            for b in range(2):
                dv, sv, rd, rs, ss = bufs[b]

                @pl.when(cs[b] < n_chunks)
                def _(b=b, dv=dv, sv=sv, ss=ss):
                    idx_cps[b] = (
                        pltpu.async_copy(dst_h.at[cs[b]], dv, ss[0]),
                        pltpu.async_copy(src_h.at[cs[b]], sv, ss[1]))
            for b in range(2):
                dv, sv, rd, rs, ss = bufs[b]

                @pl.when(cs[b] < n_chunks)
                def _(b=b, dv=dv, sv=sv, rd=rd, rs=rs, ss=ss):
                    idx_cps[b][0].wait()
                    idx_cps[b][1].wait()
                    g_cps[b] = (
                        pltpu.async_copy(table_h.at[dv], rd, ss[2]),
                        pltpu.async_copy(table_h.at[sv], rs, ss[3]))
            if has_ea:
                for b in range(2):
                    @pl.when(cs[b] < n_chunks)
                    def _(b=b):
                        pltpu.sync_copy(ea_h.at[:, cs[b]], fa)

                        def ilv(j, c2):
                            pos = (j * _L + iota) * 8
                            for f in range(dea):
                                v = fa[f, pl.ds(j * _L, _L)]
                                plsc.store_scatter(eab, [pos + f], v)
                            return c2

                        lax.fori_loop(0, K // _L, ilv, 0)
                        pltpu.sync_copy(eab, ea8_h.at[cs[b]])
            for b in range(2):
                dv, sv, rd, rs, ss = bufs[b]

                @pl.when(cs[b] < n_chunks)
                def _(b=b, rd=rd, rs=rs):
                    g_cps[b][0].wait()
                    g_cps[b][1].wait()
                    w_cps[b] = (
                        pltpu.async_copy(rd, gd_h.at[cs[b]], wsems[b][0]),
                        pltpu.async_copy(rs, gs_h.at[cs[b]], wsems[b][1]))
            for b in range(2):
                @pl.when(cs[b] < n_chunks)
                def _(b=b):
                    w_cps[b][0].wait()
                    w_cps[b][1].wait()
            return carry

        lax.fori_loop(0, n_iter, chunk, 0)

    C = n_chunks
    args = [table, src.reshape(C, K), dst.reshape(C, K)]
    if has_ea:
        args += [ea_t.reshape(dea, C, K), zeros_flat]
    return pl.kernel(
        body, out_type=tuple(out_type), mesh=mesh, scratch_types=scratch,
        compiler_params=pltpu.CompilerParams(use_tc_tiling_on_sc=False,
                                             needs_layout_passes=False),
    )(*args)


def _sc_scatter(evals, dst, zeros_nd):
    E, D = evals.shape
    NPt = zeros_nd.shape[0]
    K = 2048
    n_chunks = E // K
    n_iter = (n_chunks + _NW - 1) // _NW
    NPS = NPt // _NS
    mesh = plsc.VectorSubcoreMesh(core_axis_name="c", subcore_axis_name="s")
    scratch = [pltpu.VMEM((K,), jnp.int32),
               pltpu.VMEM((K, D), jnp.float32),
               pltpu.VMEM_SHARED((NPt, D), jnp.float32)]

    def body(ev_h, dst_h, z_h, out_h, dstv, rows, shared):
        cid = lax.axis_index("c")
        sid = lax.axis_index("s")
        wid = sid * _NC + cid
        pltpu.sync_copy(z_h.at[pl.ds(sid * NPS, NPS)],
                        shared.at[pl.ds(sid * NPS, NPS)])
        plsc.subcore_barrier()

        def chunk(i, carry):
            c = i * _NW + wid

            @pl.when(c < n_chunks)
            def _():
                pltpu.sync_copy(dst_h.at[c], dstv)
                pltpu.sync_copy(ev_h.at[c], rows)
                pltpu.sync_copy(rows, shared.at[dstv], add=True)

            return carry

        lax.fori_loop(0, n_iter, chunk, 0)
        plsc.subcore_barrier()
        pltpu.sync_copy(shared.at[pl.ds(sid * NPS, NPS)],
                        out_h.at[cid, pl.ds(sid * NPS, NPS)])

    return pl.kernel(
        body,
        out_type=jax.ShapeDtypeStruct((_NC, NPt, D), jnp.float32),
        mesh=mesh, scratch_types=scratch,
        compiler_params=pltpu.CompilerParams(use_tc_tiling_on_sc=False),
    )(evals.reshape(E // K, K, D), dst.reshape(E // K, K), zeros_nd)


def kernel(x, edge_index, edge_attr,
           in1_r1_0, in1_r1_1, in1_r1_2, in1_r1_3, in1_r1_4, in1_r1_5,
           in1_o_0, in1_o_1, in1_o_2, in1_o_3, in1_o_4, in1_o_5,
           in2_r1_0, in2_r1_1, in2_r1_2, in2_r1_3, in2_r1_4, in2_r1_5,
           in2_o_0, in2_o_1, in2_o_2, in2_o_3, in2_o_4, in2_o_5,
           r2_0, r2_1, r2_2, r2_3, r2_4, r2_5):
    in1_r1 = [in1_r1_0, in1_r1_1, in1_r1_2, in1_r1_3, in1_r1_4, in1_r1_5]
    in1_o = [in1_o_0, in1_o_1, in1_o_2, in1_o_3, in1_o_4, in1_o_5]
    in2_r1 = [in2_r1_0, in2_r1_1, in2_r1_2, in2_r1_3, in2_r1_4, in2_r1_5]
    in2_o = [in2_o_0, in2_o_1, in2_o_2, in2_o_3, in2_o_4, in2_o_5]
    r2 = [r2_0, r2_1, r2_2, r2_3, r2_4, r2_5]
    src = edge_index[0]
    dst = edge_index[1]
    N = x.shape[0]
    E = src.shape[0]

    xp = jnp.zeros((NP, 8), jnp.float32).at[:N, :3].set(x)
    z8 = jnp.zeros((NP, 8), jnp.float32)

    def g(a):
        return a.reshape(E // 16, 128)

    one = 1.0 + 0.0 * in1_r1_1[0]

    gd, gs, ea8 = _sc_gather(xp, src, dst, edge_attr.T,
                             jnp.zeros((2048 * 8,), jnp.float32))
    ea_g = ea8.reshape(E // 16, 128)
    e1 = _edge_mlp(g(gd), g(gs), ea_g, in1_r1, 3, 3, 5)
    P1 = _sc_scatter(e1.reshape(E, 8), dst, z8)
    x1p = _node_mlp(xp, P1[0], P1[1], in1_o, 3, 5, 5)

    gd, gs = _sc_gather(x1p, src, dst)
    e2 = _edge_mlp(g(gd), g(gs), e1, in2_r1, 5, 5, 7)
    P2 = _sc_scatter(e2.reshape(E, 8), dst, z8)
    x2p = _node_mlp(x1p, P2[0], P2[1], in2_o, 5, 7, 7)

    gd, gs = _sc_gather(x2p, src, dst)
    o16 = _edge_mlp(g(gd), g(gs), e2, r2, 7, 7, 1, final=True)
    return o16.T.reshape(E, 1) * one


# speedup vs baseline: 2.5016x; 1.0297x over previous
import functools

import jax
import jax.numpy as jnp
from jax import lax
from jax.experimental import pallas as pl
from jax.experimental.pallas import tpu as pltpu
from jax.experimental.pallas import tpu_sc as plsc

_NC, _NS, _L = 2, 16, 16
_NW = _NC * _NS
NP = 100352
R_BLK = 3200


def _edge_mlp_body(gd_ref, gs_ref, c_ref, w1a, w1b, w1c, b1, w2, b2, w3, b3,
                   o_ref, *, final):
    def nt(w, x):
        return lax.dot_general(w[...], x[...], (((1,), (1,)), ((), ())),
                               preferred_element_type=jnp.float32)

    h = nt(w1a, gd_ref) + nt(w1b, gs_ref) + nt(w1c, c_ref) + b1[...]
    h = jnp.maximum(h, 0.0)
    h = jnp.dot(w2[...], h, preferred_element_type=jnp.float32) + b2[...]
    h = jnp.maximum(h, 0.0)
    o = jnp.dot(w3[...], h, preferred_element_type=jnp.float32) + b3[...]
    if final:
        o_ref[...] = jax.nn.sigmoid(o)
    else:
        o_ref[...] = o.T


def _bd(w, slots_in, slots_out):
    wp = jnp.zeros((slots_in, slots_out), jnp.float32)
    wp = wp.at[:w.shape[0], :w.shape[1]].set(w)
    return jnp.kron(jnp.eye(16, dtype=jnp.float32), wp.T)


def _edge_mlp(gd, gs, c, params, da, dc, dout, *, final=False):
    W1, b1, W2, b2, W3, b3 = params
    G = gd.shape[0]
    H = W1.shape[1]
    grid = G // R_BLK
    w1a = _bd(W1[:da], 8, H)
    w1b = _bd(W1[da:2 * da], 8, H)
    w1c = _bd(W1[2 * da:], 8, H)
    w2 = jnp.kron(jnp.eye(16, dtype=jnp.float32), W2.T)
    mo = 16 if final else 128
    if final:
        w3 = jnp.kron(jnp.eye(16, dtype=jnp.float32), W3.T)
        b3v = jnp.tile(b3, 16)[:, None]
    else:
        w3 = _bd(W3, H, 8)
        b3v = jnp.tile(jnp.pad(b3, (0, 8 - dout)), 16)[:, None]
    b1v = jnp.tile(b1, 16)[:, None]
    b2v = jnp.tile(b2, 16)[:, None]

    def gm(d1):
        return pl.BlockSpec((R_BLK, d1), lambda i: (i, 0))

    def wm(shape):
        return pl.BlockSpec(shape, lambda i: (0, 0))

    if final:
        out_spec = pl.BlockSpec((16, R_BLK), lambda i: (0, i))
        out_shape = jax.ShapeDtypeStruct((16, G), jnp.float32)
    else:
        out_spec = gm(128)
        out_shape = jax.ShapeDtypeStruct((G, 128), jnp.float32)

    return pl.pallas_call(
        functools.partial(_edge_mlp_body, final=final),
        grid=(grid,),
        in_specs=[
            gm(128), gm(128), gm(128),
            wm((16 * H, 128)), wm((16 * H, 128)), wm((16 * H, 128)),
            wm((16 * H, 1)),
            wm((16 * H, 16 * H)), wm((16 * H, 1)),
            wm((mo, 16 * H)), wm((mo, 1)),
        ],
        out_specs=out_spec,
        out_shape=out_shape,
    )(gd, gs, c, w1a, w1b, w1c, b1v, w2, b2v, w3, b3v)


def _node_mlp_body(x_ref, p0_ref, p1_ref, w1a, w1b, b1, w2, b2, w3, b3,
                   o_ref, *, din_x, d_aggr, dout):
    a = x_ref[...][:, :din_x]
    g = (p0_ref[...] + p1_ref[...])[:, :d_aggr]
    h = (jnp.dot(a, w1a[...], preferred_element_type=jnp.float32)
         + jnp.dot(g, w1b[...], preferred_element_type=jnp.float32)
         + b1[...])
    h = jnp.maximum(h, 0.0)
    h = jnp.dot(h, w2[...], preferred_element_type=jnp.float32) + b2[...]
    h = jnp.maximum(h, 0.0)
    o = jnp.dot(h, w3[...], preferred_element_type=jnp.float32) + b3[...]
    o_ref[...] = jnp.concatenate(
        [o, jnp.zeros((o.shape[0], 8 - dout), jnp.float32)], axis=1)


def _node_mlp(xprev, p0, p1, params, din_x, d_aggr, dout):
    W1, b1, W2, b2, W3, b3 = params
    H = W1.shape[1]
    BN = 6272
    grid = NP // BN

    def rm(d1):
        return pl.BlockSpec((BN, d1), lambda i: (i, 0))

    def wm(shape):
        return pl.BlockSpec(shape, lambda i: (0, 0))

    return pl.pallas_call(
        functools.partial(_node_mlp_body, din_x=din_x, d_aggr=d_aggr,
                          dout=dout),
        grid=(grid,),
        in_specs=[
            rm(8), rm(8), rm(8),
            wm((din_x, H)), wm((d_aggr, H)), wm((1, H)),
            wm((H, H)), wm((1, H)),
            wm((H, dout)), wm((1, dout)),
        ],
        out_specs=rm(8),
        out_shape=jax.ShapeDtypeStruct((NP, 8), jnp.float32),
    )(xprev, p0, p1, W1[:din_x], W1[din_x:], b1[None, :],
      W2, b2[None, :], W3, b3[None, :])


def _sc_gather(table, src, dst, ea_t=None, zeros_flat=None):
    E = src.shape[0]
    Dp = table.shape[1]
    K = 2048
    n_chunks = E // K
    n_iter = (n_chunks + 2 * _NW - 1) // (2 * _NW)
    has_ea = ea_t is not None
    mesh = plsc.VectorSubcoreMesh(core_axis_name="c", subcore_axis_name="s")
    out_type = [jax.ShapeDtypeStruct((E // K, K, Dp), jnp.float32),
                jax.ShapeDtypeStruct((E // K, K, Dp), jnp.float32)]
    scratch = ([pltpu.VMEM((K,), jnp.int32)] * 4
               + [pltpu.VMEM((K, Dp), jnp.float32)] * 4
               + [pltpu.SemaphoreType.DMA] * 12)
    if has_ea:
        dea = ea_t.shape[0]
        out_type.append(jax.ShapeDtypeStruct((E // K, K * 8), jnp.float32))
        scratch += [pltpu.VMEM((dea, K), jnp.float32),
                    pltpu.VMEM((K * 8,), jnp.float32)]

    def body(*refs):
        if has_ea:
            (table_h, src_h, dst_h, ea_h, z_h, gd_h, gs_h, ea8_h,
             dv0, sv0, dv1, sv1, rd0, rs0, rd1, rs1, *rest) = refs
            sems, fa, eab = rest[:12], rest[12], rest[13]
        else:
            (table_h, src_h, dst_h, gd_h, gs_h,
             dv0, sv0, dv1, sv1, rd0, rs0, rd1, rs1, *sems) = refs
        wid = lax.axis_index("s") * _NC + lax.axis_index("c")
        iota = lax.broadcasted_iota(jnp.int32, (_L,), 0)
        bufs = ((dv0, sv0, rd0, rs0, sems[0:4]),
                (dv1, sv1, rd1, rs1, sems[4:8]))
        wsems = (sems[8:10], sems[10:12])
        if has_ea:
            pltpu.sync_copy(z_h, eab)

        def chunk(i, carry):
            cs = [(2 * i + b) * _NW + wid for b in range(2)]
            idx_cps = [None, None]
            g_cps = [None, None]
            w_cps = [None, None]
            for b in range(2):
                dv, sv, rd, rs, ss = bufs[b]

                @pl.when(cs[b] < n_chunks)
                def _(b=b, dv=dv, sv=sv, ss=ss):
                    idx_cps[b] = (
                        pltpu.async_copy(dst_h.at[cs[b]], dv, ss[0]),
                        pltpu.async_copy(src_h.at[cs[b]], sv, ss[1]))
            for b in range(2):
                dv, sv, rd, rs, ss = bufs[b]

                @pl.when(cs[b] < n_chunks)
                def _(b=b, dv=dv, sv=sv, rd=rd, rs=rs, ss=ss):
                    idx_cps[b][0].wait()
                    idx_cps[b][1].wait()
                    g_cps[b] = (
                        pltpu.async_copy(table_h.at[dv], rd, ss[2]),
                        pltpu.async_copy(table_h.at[sv], rs, ss[3]))
            if has_ea:
                for b in range(2):
                    @pl.when(cs[b] < n_chunks)
                    def _(b=b):
                        pltpu.sync_copy(ea_h.at[:, cs[b]], fa)

                        def ilv(j, c2):
                            pos = (j * _L + iota) * 8
                            for f in range(dea):
                                v = fa[f, pl.ds(j * _L, _L)]
                                plsc.store_scatter(eab, [pos + f], v)
                            return c2

                        lax.fori_loop(0, K // _L, ilv, 0)
                        pltpu.sync_copy(eab, ea8_h.at[cs[b]])
            for b in range(2):
                dv, sv, rd, rs, ss = bufs[b]

                @pl.when(cs[b] < n_chunks)
                def _(b=b, rd=rd, rs=rs):
                    g_cps[b][0].wait()
                    g_cps[b][1].wait()
                    w_cps[b] = (
                        pltpu.async_copy(rd, gd_h.at[cs[b]], wsems[b][0]),
                        pltpu.async_copy(rs, gs_h.at[cs[b]], wsems[b][1]))
            for b in range(2):
                @pl.when(cs[b] < n_chunks)
                def _(b=b):
                    w_cps[b][0].wait()
                    w_cps[b][1].wait()
            return carry

        lax.fori_loop(0, n_iter, chunk, 0)

    C = n_chunks
    args = [table, src.reshape(C, K), dst.reshape(C, K)]
    if has_ea:
        args += [ea_t.reshape(dea, C, K), zeros_flat]
    return pl.kernel(
        body, out_type=tuple(out_type), mesh=mesh, scratch_types=scratch,
        compiler_params=pltpu.CompilerParams(use_tc_tiling_on_sc=False,
                                             needs_layout_passes=False),
    )(*args)


def _sc_scatter(evals, dst, zeros_nd):
    E, D = evals.shape
    NPt = zeros_nd.shape[0]
    K = 2048
    n_chunks = E // K
    n_iter = (n_chunks + 2 * _NW - 1) // (2 * _NW)
    NPS = NPt // _NS
    mesh = plsc.VectorSubcoreMesh(core_axis_name="c", subcore_axis_name="s")
    scratch = ([pltpu.VMEM((K,), jnp.int32)] * 2
               + [pltpu.VMEM((K, D), jnp.float32)] * 2
               + [pltpu.VMEM_SHARED((NPt, D), jnp.float32)]
               + [pltpu.SemaphoreType.DMA] * 4)

    def body(ev_h, dst_h, z_h, out_h, dv0, dv1, rw0, rw1, shared, *sems):
        cid = lax.axis_index("c")
        sid = lax.axis_index("s")
        wid = sid * _NC + cid
        pltpu.sync_copy(z_h.at[pl.ds(sid * NPS, NPS)],
                        shared.at[pl.ds(sid * NPS, NPS)])
        plsc.subcore_barrier()

        bufs = ((dv0, rw0, sems[0:2]), (dv1, rw1, sems[2:4]))

        def chunk(i, carry):
            cs = [(2 * i + b) * _NW + wid for b in range(2)]
            cps = [None, None]
            for b in range(2):
                dv, rw, ss = bufs[b]

                @pl.when(cs[b] < n_chunks)
                def _(b=b, dv=dv, rw=rw, ss=ss):
                    cps[b] = (
                        pltpu.async_copy(dst_h.at[cs[b]], dv, ss[0]),
                        pltpu.async_copy(ev_h.at[cs[b]], rw, ss[1]))
            for b in range(2):
                dv, rw, ss = bufs[b]

                @pl.when(cs[b] < n_chunks)
                def _(b=b, dv=dv, rw=rw):
                    cps[b][0].wait()
                    cps[b][1].wait()
                    pltpu.sync_copy(rw, shared.at[dv], add=True)
            return carry

        lax.fori_loop(0, n_iter, chunk, 0)
        plsc.subcore_barrier()
        pltpu.sync_copy(shared.at[pl.ds(sid * NPS, NPS)],
                        out_h.at[cid, pl.ds(sid * NPS, NPS)])

    return pl.kernel(
        body,
        out_type=jax.ShapeDtypeStruct((_NC, NPt, D), jnp.float32),
        mesh=mesh, scratch_types=scratch,
        compiler_params=pltpu.CompilerParams(use_tc_tiling_on_sc=False),
    )(evals.reshape(E // K, K, D), dst.reshape(E // K, K), zeros_nd)


def kernel(x, edge_index, edge_attr,
           in1_r1_0, in1_r1_1, in1_r1_2, in1_r1_3, in1_r1_4, in1_r1_5,
           in1_o_0, in1_o_1, in1_o_2, in1_o_3, in1_o_4, in1_o_5,
           in2_r1_0, in2_r1_1, in2_r1_2, in2_r1_3, in2_r1_4, in2_r1_5,
           in2_o_0, in2_o_1, in2_o_2, in2_o_3, in2_o_4, in2_o_5,
           r2_0, r2_1, r2_2, r2_3, r2_4, r2_5):
    in1_r1 = [in1_r1_0, in1_r1_1, in1_r1_2, in1_r1_3, in1_r1_4, in1_r1_5]
    in1_o = [in1_o_0, in1_o_1, in1_o_2, in1_o_3, in1_o_4, in1_o_5]
    in2_r1 = [in2_r1_0, in2_r1_1, in2_r1_2, in2_r1_3, in2_r1_4, in2_r1_5]
    in2_o = [in2_o_0, in2_o_1, in2_o_2, in2_o_3, in2_o_4, in2_o_5]
    r2 = [r2_0, r2_1, r2_2, r2_3, r2_4, r2_5]
    src = edge_index[0]
    dst = edge_index[1]
    N = x.shape[0]
    E = src.shape[0]

    xp = jnp.zeros((NP, 8), jnp.float32).at[:N, :3].set(x)
    z8 = jnp.zeros((NP, 8), jnp.float32)

    def g(a):
        return a.reshape(E // 16, 128)

    one = 1.0 + 0.0 * in1_r1_1[0]

    gd, gs, ea8 = _sc_gather(xp, src, dst, edge_attr.T,
                             jnp.zeros((2048 * 8,), jnp.float32))
    ea_g = ea8.reshape(E // 16, 128)
    e1 = _edge_mlp(g(gd), g(gs), ea_g, in1_r1, 3, 3, 5)
    P1 = _sc_scatter(e1.reshape(E, 8), dst, z8)
    x1p = _node_mlp(xp, P1[0], P1[1], in1_o, 3, 5, 5)

    gd, gs = _sc_gather(x1p, src, dst)
    e2 = _edge_mlp(g(gd), g(gs), e1, in2_r1, 5, 5, 7)
    P2 = _sc_scatter(e2.reshape(E, 8), dst, z8)
    x2p = _node_mlp(x1p, P2[0], P2[1], in2_o, 5, 7, 7)

    gd, gs = _sc_gather(x2p, src, dst)
    o16 = _edge_mlp(g(gd), g(gs), e2, r2, 7, 7, 1, final=True)
    return o16.T.reshape(E, 1) * one
